# Initial kernel scaffold; baseline (speedup 1.0000x reference)
import jax
import jax.numpy as jnp
from jax.experimental import pallas as pl


def kernel(src_feat, src_cluster, src_idx, tgt_feat, tgt_cluster, W1, b1, gamma, beta, W2, b2, src_cluster_labels, src_cluster_centers, tgt_cluster_centers):
    raise NotImplementedError("write your pallas kernel here")


# speedup vs baseline: 1.0085x; 1.0085x over previous
import functools

import jax
import jax.numpy as jnp
from jax import lax
from jax.experimental import pallas as pl
from jax.experimental.pallas import tpu as pltpu
from jax.experimental.pallas import tpu_sc as plsc

D = 256
H = 128
K = 1024
N = 16384

_EXT_CHUNK = 512
_SEG_CHUNK = 128
_NUM_TILES = 16
_NUM_WORKERS = 32


def _bf16_round(x):
    return x.astype(jnp.bfloat16).astype(jnp.float32)


def _extract_body(x_ref, w1_ref, b1_ref, g_ref, be_ref, w2_ref, b2_ref,
                  out_ref, h_ref):
    nchunks = N // _EXT_CHUNK
    w1b = w1_ref[...].astype(jnp.bfloat16)
    w2b = w2_ref[...].astype(jnp.bfloat16)

    def pass1(i, _):
        xc = x_ref[pl.ds(i * _EXT_CHUNK, _EXT_CHUNK), :].astype(jnp.bfloat16)
        h = lax.dot_general(xc, w1b, (((1,), (1,)), ((), ())),
                            preferred_element_type=jnp.float32)
        h = jnp.maximum(h + b1_ref[...], 0.0)
        h_ref[pl.ds(i * _EXT_CHUNK, _EXT_CHUNK), :] = h
        return 0

    lax.fori_loop(0, nchunks, pass1, 0, unroll=False)
    mu = jnp.sum(h_ref[...], axis=0, keepdims=True) * (1.0 / N)
    d1 = h_ref[0:N // 2, :] - mu
    d2 = h_ref[N // 2:N, :] - mu
    va = jnp.sum(d1 * d1, axis=0, keepdims=True)
    vb = jnp.sum(d2 * d2, axis=0, keepdims=True)
    var = (va + vb) * (1.0 / N)
    denom = jnp.sqrt(var + 1e-5)

    def pass2(i, _):
        h = h_ref[pl.ds(i * _EXT_CHUNK, _EXT_CHUNK), :]
        hn = (h - mu) / denom * g_ref[...] + be_ref[...]
        o = lax.dot_general(hn.astype(jnp.bfloat16), w2b,
                            (((1,), (1,)), ((), ())),
                            preferred_element_type=jnp.float32)
        o = o + b2_ref[...]
        out_ref[pl.ds(i * _EXT_CHUNK, _EXT_CHUNK), :] = _bf16_round(o)
        return 0

    lax.fori_loop(0, nchunks, pass2, 0)


_extract_call = pl.pallas_call(
    _extract_body,
    out_shape=jax.ShapeDtypeStruct((N, D), jnp.float32),
    scratch_shapes=[pltpu.VMEM((N, H), jnp.float32)],
)


_sc_mesh = plsc.VectorSubcoreMesh(core_axis_name="c", subcore_axis_name="s")

_RG = 4
_CG = 4
_CW = D // _CG
_ROWS_PER_RG = N // _RG
_SEG_CH = 256


@functools.partial(
    pl.kernel,
    mesh=_sc_mesh,
    out_type=(jax.ShapeDtypeStruct((2, _RG, _CG, K, _CW), jnp.float32),
              jax.ShapeDtypeStruct((2, _RG, K, 16), jnp.float32)),
    scratch_types=[
        pltpu.VMEM((_SEG_CH, _CW), jnp.float32),
        pltpu.VMEM((_SEG_CH,), jnp.int32),
        pltpu.VMEM((K, _CW), jnp.float32),
        pltpu.VMEM((K, 16), jnp.float32),
    ],
    compiler_params=pltpu.CompilerParams(needs_layout_passes=False,
                                         use_tc_tiling_on_sc=False),
)
def _segsum_call(src_f, tgt_f, src_c, tgt_c, zeros_kc, zeros_k16,
                 sums_out, cnts_out, rows_v, ids_v, acc_v, cnt_v):
    c = lax.axis_index("c")
    s = lax.axis_index("s")
    rg = s // _CG
    cg = s % _CG
    col0 = cg * _CW

    pltpu.sync_copy(zeros_kc, acc_v)
    pltpu.sync_copy(zeros_k16, cnt_v)

    iota = lax.broadcasted_iota(jnp.int32, (16,), 0)
    ones16 = jnp.ones((16,), jnp.float32)

    def _accumulate(feat, clus):
        nchunk = _ROWS_PER_RG // _SEG_CH

        def chunk_body(k, _):
            row0 = rg * _ROWS_PER_RG + k * _SEG_CH
            pltpu.sync_copy(feat.at[pl.ds(row0, _SEG_CH), pl.ds(col0, _CW)],
                            rows_v)
            pltpu.sync_copy(clus.at[pl.ds(row0, _SEG_CH)], ids_v)

            def row_body(r, _):
                id16 = plsc.load_gather(ids_v, [jnp.full((16,), r, jnp.int32)])
                for j in range(_CW // 16):
                    val = rows_v[r, pl.ds(j * 16, 16)]
                    plsc.addupdate_scatter(acc_v, [id16, j * 16 + iota], val)
                return 0

            lax.fori_loop(0, _SEG_CH, row_body, 0)

            @pl.when(cg == 0)
            def _():
                def cnt_body(g, _):
                    id16 = ids_v[pl.ds(g * 16, 16)]
                    plsc.addupdate_scatter(cnt_v, [id16, iota], ones16)
                    return 0
                lax.fori_loop(0, _SEG_CH // 16, cnt_body, 0)

            return 0

        lax.fori_loop(0, nchunk, chunk_body, 0)

    @pl.when(c == 0)
    def _():
        _accumulate(src_f, src_c)

    @pl.when(c == 1)
    def _():
        _accumulate(tgt_f, tgt_c)

    pltpu.sync_copy(acc_v, sums_out.at[c, rg, cg])

    @pl.when(cg == 0)
    def _():
        pltpu.sync_copy(cnt_v, cnts_out.at[c, rg])


def _norm_body(sums_ref, cnt_ref, prev_ref, out_ref):
    s4 = jnp.sum(sums_ref[0], axis=0)
    cnt = jnp.sum(jnp.sum(cnt_ref[0], axis=0), axis=1, keepdims=True) + 1e-6
    coef = _bf16_round(1.0 / cnt + 1.0)
    new = _bf16_round(s4) * coef[None]
    upd = 0.5 * prev_ref[0] + 0.5 * new
    nrm = jnp.sqrt(jnp.sum(upd * upd, axis=(0, 2)))
    out_ref[0] = upd / jnp.maximum(nrm, 1e-12)[None, :, None]


_norm_call = pl.pallas_call(
    _norm_body,
    grid=(2, 8),
    in_specs=[
        pl.BlockSpec((1, _RG, _CG, K // 8, _CW), lambda a, b: (a, 0, 0, b, 0)),
        pl.BlockSpec((1, _RG, K // 8, 16), lambda a, b: (a, 0, b, 0)),
        pl.BlockSpec((1, _CG, K // 8, _CW), lambda a, b: (a, 0, b, 0)),
    ],
    out_specs=pl.BlockSpec((1, _CG, K // 8, _CW), lambda a, b: (a, 0, b, 0)),
    out_shape=jax.ShapeDtypeStruct((2, _CG, K, _CW), jnp.float32),
)


def _sim_body(tn_ref, sn_ref, out_ref):
    sim = jnp.zeros((K // 8, K), jnp.float32)
    for g in range(_CG):
        sim = sim + lax.dot_general(
            tn_ref[0, g].astype(jnp.bfloat16),
            sn_ref[0, g].astype(jnp.bfloat16),
            (((1,), (1,)), ((), ())),
            preferred_element_type=jnp.float32)
    mx = jnp.max(sim, axis=1, keepdims=True)
    ids = lax.broadcasted_iota(jnp.int32, sim.shape, 1)
    cand = jnp.where(sim == mx, ids, K)
    out_ref[...] = jnp.min(cand, axis=1, keepdims=True)


_sim_call = pl.pallas_call(
    _sim_body,
    grid=(8,),
    in_specs=[
        pl.BlockSpec((1, _CG, K // 8, _CW), lambda i: (1, 0, i, 0)),
        pl.BlockSpec((1, _CG, K, _CW), lambda i: (0, 0, 0, 0)),
    ],
    out_specs=pl.BlockSpec((K // 8, 1), lambda i: (i, 0)),
    out_shape=jax.ShapeDtypeStruct((K, 1), jnp.int32),
)


@functools.partial(
    pl.kernel,
    mesh=_sc_mesh,
    out_type=jax.ShapeDtypeStruct((N,), jnp.int32),
    scratch_types=[
        pltpu.VMEM((K,), jnp.int32),
        pltpu.VMEM((K,), jnp.int32),
        pltpu.VMEM((N // _NUM_WORKERS,), jnp.int32),
        pltpu.VMEM((N // _NUM_WORKERS,), jnp.int32),
    ],
    compiler_params=pltpu.CompilerParams(needs_layout_passes=False,
                                         use_tc_tiling_on_sc=False),
)
def _gather_call(top_hbm, lab_hbm, tc_hbm, out_hbm, top_v, lab_v, idx_v, out_v):
    c = lax.axis_index("c")
    s = lax.axis_index("s")
    wid = s * 2 + c
    npt = N // _NUM_WORKERS
    pltpu.sync_copy(top_hbm, top_v)
    pltpu.sync_copy(lab_hbm, lab_v)
    pltpu.sync_copy(tc_hbm.at[pl.ds(wid * npt, npt)], idx_v)

    def body(j, _):
        i16 = idx_v[pl.ds(j * 16, 16)]
        t16 = plsc.load_gather(top_v, [i16])
        l16 = plsc.load_gather(lab_v, [t16])
        out_v[pl.ds(j * 16, 16)] = l16
        return 0

    lax.fori_loop(0, npt // 16, body, 0)
    pltpu.sync_copy(out_v, out_hbm.at[pl.ds(wid * npt, npt)])


def kernel(src_feat, src_cluster, src_idx, tgt_feat, tgt_cluster, W1, b1,
           gamma, beta, W2, b2, src_cluster_labels, src_cluster_centers,
           tgt_cluster_centers):
    src_c = src_cluster.astype(jnp.int32)
    tgt_c = tgt_cluster.astype(jnp.int32)
    b1r = b1.reshape(1, H)
    gr = gamma.reshape(1, H)
    ber = beta.reshape(1, H)
    b2r = b2.reshape(1, D)

    sf = _extract_call(src_feat, W1, b1r, gr, ber, W2, b2r)
    tf = _extract_call(tgt_feat, W1, b1r, gr, ber, W2, b2r)

    zeros_kc = jnp.zeros((K, _CW), jnp.float32)
    zeros_k16 = jnp.zeros((K, 16), jnp.float32)
    sums, cnts = _segsum_call(sf, tf, src_c, tgt_c, zeros_kc, zeros_k16)

    prev_src = lax.dynamic_index_in_dim(src_cluster_centers, src_idx, 0,
                                        keepdims=False)
    prev = jnp.stack([prev_src, tgt_cluster_centers])
    prev = prev.reshape(2, K, _CG, _CW).transpose(0, 2, 1, 3)
    normed = _norm_call(sums, cnts, prev)

    top = _sim_call(normed, normed)
    labels = lax.dynamic_index_in_dim(src_cluster_labels, src_idx, 0,
                                      keepdims=False).astype(jnp.int32)
    return _gather_call(top.reshape(K), labels, tgt_c)


# speedup vs baseline: 1.0852x; 1.0760x over previous
import functools

import jax
import jax.numpy as jnp
from jax import lax
from jax.experimental import pallas as pl
from jax.experimental.pallas import tpu as pltpu
from jax.experimental.pallas import tpu_sc as plsc

D = 256
H = 128
K = 1024
N = 16384

_EXT_CHUNK = 512
_SEG_CHUNK = 128
_NUM_TILES = 16
_NUM_WORKERS = 32


def _bf16_round(x):
    return x.astype(jnp.bfloat16).astype(jnp.float32)


def _extract_body(x_ref, w1_ref, b1_ref, g_ref, be_ref, w2_ref, b2_ref,
                  out_ref, h_ref):
    nchunks = N // _EXT_CHUNK
    w1b = w1_ref[...].astype(jnp.bfloat16)
    w2b = w2_ref[...].astype(jnp.bfloat16)

    def pass1(i, _):
        xc = x_ref[pl.ds(i * _EXT_CHUNK, _EXT_CHUNK), :].astype(jnp.bfloat16)
        h = lax.dot_general(xc, w1b, (((1,), (1,)), ((), ())),
                            preferred_element_type=jnp.float32)
        h = jnp.maximum(h + b1_ref[...], 0.0)
        h_ref[pl.ds(i * _EXT_CHUNK, _EXT_CHUNK), :] = h
        return 0

    lax.fori_loop(0, nchunks, pass1, 0, unroll=False)
    mu = jnp.sum(h_ref[...], axis=0, keepdims=True) * (1.0 / N)
    d1 = h_ref[0:N // 2, :] - mu
    d2 = h_ref[N // 2:N, :] - mu
    va = jnp.sum(d1 * d1, axis=0, keepdims=True)
    vb = jnp.sum(d2 * d2, axis=0, keepdims=True)
    var = (va + vb) * (1.0 / N)
    denom = jnp.sqrt(var + 1e-5)

    def pass2(i, _):
        h = h_ref[pl.ds(i * _EXT_CHUNK, _EXT_CHUNK), :]
        hn = (h - mu) / denom * g_ref[...] + be_ref[...]
        o = lax.dot_general(hn.astype(jnp.bfloat16), w2b,
                            (((1,), (1,)), ((), ())),
                            preferred_element_type=jnp.float32)
        o = o + b2_ref[...]
        out_ref[pl.ds(i * _EXT_CHUNK, _EXT_CHUNK), :] = _bf16_round(o)
        return 0

    lax.fori_loop(0, nchunks, pass2, 0)


_extract_call = pl.pallas_call(
    _extract_body,
    out_shape=jax.ShapeDtypeStruct((N, D), jnp.float32),
    scratch_shapes=[pltpu.VMEM((N, H), jnp.float32)],
)


_sc_mesh = plsc.VectorSubcoreMesh(core_axis_name="c", subcore_axis_name="s")

_RG = 4
_CG = 4
_CW = D // _CG
_ROWS_PER_RG = N // _RG
_SEG_CH = 256


@functools.partial(
    pl.kernel,
    mesh=_sc_mesh,
    out_type=(jax.ShapeDtypeStruct((2, _RG, _CG, K, _CW), jnp.float32),
              jax.ShapeDtypeStruct((2, _RG, K, 16), jnp.float32)),
    scratch_types=[
        pltpu.VMEM((_SEG_CH, _CW), jnp.float32),
        pltpu.VMEM((_SEG_CH, _CW), jnp.float32),
        pltpu.VMEM((_SEG_CH,), jnp.int32),
        pltpu.VMEM((_SEG_CH,), jnp.int32),
        pltpu.VMEM((K, _CW), jnp.float32),
        pltpu.VMEM((K, 16), jnp.float32),
        pltpu.SemaphoreType.DMA,
        pltpu.SemaphoreType.DMA,
        pltpu.SemaphoreType.DMA,
        pltpu.SemaphoreType.DMA,
    ],
    compiler_params=pltpu.CompilerParams(needs_layout_passes=False,
                                         use_tc_tiling_on_sc=False),
)
def _segsum_call(src_f, tgt_f, src_c, tgt_c, zeros_kc, zeros_k16,
                 sums_out, cnts_out, rows_v0, rows_v1, ids_v0, ids_v1,
                 acc_v, cnt_v, sem_r0, sem_r1, sem_i0, sem_i1):
    c = lax.axis_index("c")
    s = lax.axis_index("s")
    rg = s // _CG
    cg = s % _CG
    col0 = cg * _CW

    pltpu.sync_copy(zeros_kc, acc_v)
    pltpu.sync_copy(zeros_k16, cnt_v)

    iota = lax.broadcasted_iota(jnp.int32, (16,), 0)
    ones16 = jnp.ones((16,), jnp.float32)

    bufs = [(rows_v0, ids_v0, sem_r0, sem_i0),
            (rows_v1, ids_v1, sem_r1, sem_i1)]

    def _accumulate(feat, clus):
        nchunk = _ROWS_PER_RG // _SEG_CH

        def _start(k):
            rv, iv, sr, si = bufs[k % 2]
            row0 = rg * _ROWS_PER_RG + k * _SEG_CH
            d1 = pltpu.make_async_copy(
                feat.at[pl.ds(row0, _SEG_CH), pl.ds(col0, _CW)], rv, sr)
            d2 = pltpu.make_async_copy(clus.at[pl.ds(row0, _SEG_CH)], iv, si)
            d1.start()
            d2.start()
            return d1, d2

        def _process(k):
            rv, iv, _, _ = bufs[k % 2]

            def row_body(r, _):
                id16 = plsc.load_gather(iv, [jnp.full((16,), r, jnp.int32)])
                for j in range(_CW // 16):
                    val = rv[r, pl.ds(j * 16, 16)]
                    plsc.addupdate_scatter(acc_v, [id16, j * 16 + iota], val)
                return 0

            lax.fori_loop(0, _SEG_CH, row_body, 0)

            @pl.when(cg == 0)
            def _():
                def cnt_body(g, _):
                    id16 = iv[pl.ds(g * 16, 16)]
                    plsc.addupdate_scatter(cnt_v, [id16, iota], ones16)
                    return 0
                lax.fori_loop(0, _SEG_CH // 16, cnt_body, 0)

        pend = _start(0)
        for k in range(nchunk):
            if k + 1 < nchunk:
                nxt = _start(k + 1)
            pend[0].wait()
            pend[1].wait()
            _process(k)
            if k + 1 < nchunk:
                pend = nxt

    @pl.when(c == 0)
    def _():
        _accumulate(src_f, src_c)

    @pl.when(c == 1)
    def _():
        _accumulate(tgt_f, tgt_c)

    pltpu.sync_copy(acc_v, sums_out.at[c, rg, cg])

    @pl.when(cg == 0)
    def _():
        pltpu.sync_copy(cnt_v, cnts_out.at[c, rg])


def _norm_body(sums_ref, cnt_ref, prev_ref, out_ref):
    s4 = jnp.sum(sums_ref[0], axis=0)
    cnt = jnp.sum(jnp.sum(cnt_ref[0], axis=0), axis=1, keepdims=True) + 1e-6
    coef = _bf16_round(1.0 / cnt + 1.0)
    new = _bf16_round(s4) * coef[None]
    upd = 0.5 * prev_ref[0] + 0.5 * new
    nrm = jnp.sqrt(jnp.sum(upd * upd, axis=(0, 2)))
    out_ref[0] = upd / jnp.maximum(nrm, 1e-12)[None, :, None]


_norm_call = pl.pallas_call(
    _norm_body,
    grid=(2, 8),
    in_specs=[
        pl.BlockSpec((1, _RG, _CG, K // 8, _CW), lambda a, b: (a, 0, 0, b, 0)),
        pl.BlockSpec((1, _RG, K // 8, 16), lambda a, b: (a, 0, b, 0)),
        pl.BlockSpec((1, _CG, K // 8, _CW), lambda a, b: (a, 0, b, 0)),
    ],
    out_specs=pl.BlockSpec((1, _CG, K // 8, _CW), lambda a, b: (a, 0, b, 0)),
    out_shape=jax.ShapeDtypeStruct((2, _CG, K, _CW), jnp.float32),
)


def _sim_body(tn_ref, sn_ref, out_ref):
    sim = jnp.zeros((K // 8, K), jnp.float32)
    for g in range(_CG):
        sim = sim + lax.dot_general(
            tn_ref[0, g].astype(jnp.bfloat16),
            sn_ref[0, g].astype(jnp.bfloat16),
            (((1,), (1,)), ((), ())),
            preferred_element_type=jnp.float32)
    mx = jnp.max(sim, axis=1, keepdims=True)
    ids = lax.broadcasted_iota(jnp.int32, sim.shape, 1)
    cand = jnp.where(sim == mx, ids, K)
    out_ref[...] = jnp.min(cand, axis=1, keepdims=True)


_sim_call = pl.pallas_call(
    _sim_body,
    grid=(8,),
    in_specs=[
        pl.BlockSpec((1, _CG, K // 8, _CW), lambda i: (1, 0, i, 0)),
        pl.BlockSpec((1, _CG, K, _CW), lambda i: (0, 0, 0, 0)),
    ],
    out_specs=pl.BlockSpec((K // 8, 1), lambda i: (i, 0)),
    out_shape=jax.ShapeDtypeStruct((K, 1), jnp.int32),
)


@functools.partial(
    pl.kernel,
    mesh=_sc_mesh,
    out_type=jax.ShapeDtypeStruct((N,), jnp.int32),
    scratch_types=[
        pltpu.VMEM((K,), jnp.int32),
        pltpu.VMEM((K,), jnp.int32),
        pltpu.VMEM((N // _NUM_WORKERS,), jnp.int32),
        pltpu.VMEM((N // _NUM_WORKERS,), jnp.int32),
    ],
    compiler_params=pltpu.CompilerParams(needs_layout_passes=False,
                                         use_tc_tiling_on_sc=False),
)
def _gather_call(top_hbm, lab_hbm, tc_hbm, out_hbm, top_v, lab_v, idx_v, out_v):
    c = lax.axis_index("c")
    s = lax.axis_index("s")
    wid = s * 2 + c
    npt = N // _NUM_WORKERS
    pltpu.sync_copy(top_hbm, top_v)
    pltpu.sync_copy(lab_hbm, lab_v)
    pltpu.sync_copy(tc_hbm.at[pl.ds(wid * npt, npt)], idx_v)

    def body(j, _):
        i16 = idx_v[pl.ds(j * 16, 16)]
        t16 = plsc.load_gather(top_v, [i16])
        l16 = plsc.load_gather(lab_v, [t16])
        out_v[pl.ds(j * 16, 16)] = l16
        return 0

    lax.fori_loop(0, npt // 16, body, 0)
    pltpu.sync_copy(out_v, out_hbm.at[pl.ds(wid * npt, npt)])


def kernel(src_feat, src_cluster, src_idx, tgt_feat, tgt_cluster, W1, b1,
           gamma, beta, W2, b2, src_cluster_labels, src_cluster_centers,
           tgt_cluster_centers):
    src_c = src_cluster.astype(jnp.int32)
    tgt_c = tgt_cluster.astype(jnp.int32)
    b1r = b1.reshape(1, H)
    gr = gamma.reshape(1, H)
    ber = beta.reshape(1, H)
    b2r = b2.reshape(1, D)

    sf = _extract_call(src_feat, W1, b1r, gr, ber, W2, b2r)
    tf = _extract_call(tgt_feat, W1, b1r, gr, ber, W2, b2r)

    zeros_kc = jnp.zeros((K, _CW), jnp.float32)
    zeros_k16 = jnp.zeros((K, 16), jnp.float32)
    sums, cnts = _segsum_call(sf, tf, src_c, tgt_c, zeros_kc, zeros_k16)

    prev_src = lax.dynamic_index_in_dim(src_cluster_centers, src_idx, 0,
                                        keepdims=False)
    prev = jnp.stack([prev_src, tgt_cluster_centers])
    prev = prev.reshape(2, K, _CG, _CW).transpose(0, 2, 1, 3)
    normed = _norm_call(sums, cnts, prev)

    top = _sim_call(normed, normed)
    labels = lax.dynamic_index_in_dim(src_cluster_labels, src_idx, 0,
                                      keepdims=False).astype(jnp.int32)
    return _gather_call(top.reshape(K), labels, tgt_c)
